# baseline (speedup 1.0000x reference)
import functools

import jax
import jax.numpy as jnp
from jax import lax
from jax.experimental import pallas as pl
from jax.experimental.pallas import tpu as pltpu
from jax.experimental.pallas import tpu_sc as plsc


def _node_kernel(feat_ref, mass_ref, W1n_ref, b1n_ref, W2n_ref, b2n_ref,
                 gn_ref, bn_ref, latent_ref, ke_ref):
    feat = feat_ref[...]
    pos = feat[:, 0:3]
    v = feat[:, 3:6]
    mass = mass_ref[...]
    W1n = W1n_ref[...]
    h1 = jnp.maximum(pos[:, 0:1] * W1n[0:1, :] + pos[:, 1:2] * W1n[1:2, :]
                     + pos[:, 2:3] * W1n[2:3, :] + b1n_ref[...], 0.0)
    h = jnp.dot(h1, W2n_ref[...], preferred_element_type=jnp.float32) + b2n_ref[...]
    mu = jnp.mean(h, axis=-1, keepdims=True)
    var = jnp.mean((h - mu) ** 2, axis=-1, keepdims=True)
    latent_ref[...] = (h - mu) / jnp.sqrt(var + 1e-5) * gn_ref[...] + bn_ref[...]
    ke_part = 0.5 * jnp.sum(mass * jnp.sum(v * v, axis=-1, keepdims=True), keepdims=True)

    @pl.when(pl.program_id(0) == 0)
    def _():
        ke_ref[...] = jnp.zeros_like(ke_ref)

    ke_ref[...] += ke_part


def _node_stage(feat, mass, W1n, b1n, W2n, b2n, gn, bn, bn_rows):
    n = feat.shape[0]
    grid = (n // bn_rows,)
    latent, ke = pl.pallas_call(
        _node_kernel,
        grid=grid,
        in_specs=[
            pl.BlockSpec((bn_rows, 6), lambda i: (i, 0)),
            pl.BlockSpec((bn_rows, 1), lambda i: (i, 0)),
            pl.BlockSpec((3, 128), lambda i: (0, 0)),
            pl.BlockSpec((1, 128), lambda i: (0, 0)),
            pl.BlockSpec((128, 128), lambda i: (0, 0)),
            pl.BlockSpec((1, 128), lambda i: (0, 0)),
            pl.BlockSpec((1, 128), lambda i: (0, 0)),
            pl.BlockSpec((1, 128), lambda i: (0, 0)),
        ],
        out_specs=[
            pl.BlockSpec((bn_rows, 128), lambda i: (i, 0)),
            pl.BlockSpec((1, 1), lambda i: (0, 0)),
        ],
        out_shape=[
            jax.ShapeDtypeStruct((n, 128), jnp.float32),
            jax.ShapeDtypeStruct((1, 1), jnp.float32),
        ],
    )(feat, mass, W1n, b1n.reshape(1, 128), W2n, b2n.reshape(1, 128),
      gn.reshape(1, 128), bn.reshape(1, 128))
    return latent, ke


def _sc_geometry():
    try:
        info = plsc.get_sparse_core_info()
        return info.num_cores, info.num_subcores
    except Exception:
        return 2, 16


def _gather_sum(latent, idx_flat, e_pad, ce):
    nc, ns = _sc_geometry()
    nw = nc * ns
    e_per_w = e_pad // nw
    n_chunks = e_per_w // ce
    mesh = plsc.VectorSubcoreMesh(core_axis_name="c", subcore_axis_name="s",
                                  num_cores=nc, num_subcores=ns)

    nbuf = 4
    @functools.partial(
        pl.kernel, mesh=mesh,
        out_type=jax.ShapeDtypeStruct((e_pad, 128), jnp.float32),
        scratch_types=[
            pltpu.VMEM((n_chunks * 4 * ce,), jnp.int32),
            [pltpu.VMEM((4 * ce, 128), jnp.float32)] * nbuf,
            [pltpu.VMEM((ce, 128), jnp.float32)] * nbuf,
            [pltpu.SemaphoreType.DMA] * nbuf,
            [pltpu.SemaphoreType.DMA] * nbuf,
        ],
    )
    def k(table_hbm, idx_hbm, out_hbm, idx_all, rows, outs, gsem, ssem):
        wid = lax.axis_index("s") * nc + lax.axis_index("c")
        cb = wid * n_chunks
        pltpu.sync_copy(idx_hbm.at[pl.ds(cb * 4 * ce, n_chunks * 4 * ce)],
                        idx_all)
        for j in range(nbuf - 1):
            pltpu.async_copy(
                table_hbm.at[idx_all.at[pl.ds(j * 4 * ce, 4 * ce)]],
                rows[j], gsem[j])

        def group_body(g, _):
            for j in range(nbuf):
                ci = nbuf * g + j
                r_cur, gs_cur = rows[j], gsem[j]
                jp = (j + nbuf - 1) % nbuf
                r_pre, gs_pre = rows[jp], gsem[jp]
                o_cur, ss_cur = outs[j], ssem[j]

                @pl.when(ci + nbuf - 1 < n_chunks)
                def _():
                    pltpu.async_copy(
                        table_hbm.at[idx_all.at[pl.ds((ci + nbuf - 1) * 4 * ce,
                                                      4 * ce)]],
                        r_pre, gs_pre)

                pltpu.make_async_copy(
                    table_hbm.at[idx_all.at[pl.ds(ci * 4 * ce, 4 * ce)]],
                    r_cur, gs_cur).wait()

                @pl.when(ci >= nbuf)
                def _():
                    pltpu.make_async_copy(o_cur, out_hbm.at[pl.ds(0, ce)],
                                          ss_cur).wait()

                @plsc.parallel_loop(0, ce, unroll=8)
                def _(ei):
                    b = 4 * ei
                    for c in range(8):
                        col = pl.ds(c * 16, 16)
                        o_cur[ei, col] = (r_cur[b, col] + r_cur[b + 1, col]
                                          + r_cur[b + 2, col]
                                          + r_cur[b + 3, col])

                pltpu.async_copy(o_cur, out_hbm.at[pl.ds((cb + ci) * ce, ce)],
                                 ss_cur)
            return 0

        lax.fori_loop(0, n_chunks // nbuf, group_body, 0)
        for j in range(nbuf):
            pltpu.make_async_copy(outs[j], out_hbm.at[pl.ds(0, ce)],
                                  ssem[j]).wait()

    return k(latent, idx_flat)


def _elem_kernel(coords_ref, mat_ref, W1c_ref, w1m_ref, b1e_ref, W2e_ref,
                 b2e_ref, ge_ref, be_ref, ie_ref):
    coords = coords_ref[...]
    mat = mat_ref[...]
    h1 = jnp.maximum(jnp.dot(coords, W1c_ref[...], preferred_element_type=jnp.float32)
                     + mat * w1m_ref[...] + b1e_ref[...], 0.0)
    he = jnp.sum(h1 * W2e_ref[...], axis=-1, keepdims=True) + b2e_ref[...]
    mu = jnp.mean(he, axis=-1, keepdims=True)
    var = jnp.mean((he - mu) ** 2, axis=-1, keepdims=True)
    pe = (he - mu) / jnp.sqrt(var + 1e-5) * ge_ref[...] + be_ref[...]
    part = jnp.sum(pe, keepdims=True)

    @pl.when(pl.program_id(0) == 0)
    def _():
        ie_ref[...] = jnp.zeros_like(ie_ref)

    ie_ref[...] += part


def _elem_stage(coords, mat, W1e, b1e, W2e, b2e, ge, be, be_rows):
    e = coords.shape[0]
    grid = (e // be_rows,)
    W1c = W1e[:128, :]
    w1m = W1e[128:129, :]
    ie = pl.pallas_call(
        _elem_kernel,
        grid=grid,
        in_specs=[
            pl.BlockSpec((be_rows, 128), lambda i: (i, 0)),
            pl.BlockSpec((be_rows, 1), lambda i: (i, 0)),
            pl.BlockSpec((128, 128), lambda i: (0, 0)),
            pl.BlockSpec((1, 128), lambda i: (0, 0)),
            pl.BlockSpec((1, 128), lambda i: (0, 0)),
            pl.BlockSpec((1, 128), lambda i: (0, 0)),
            pl.BlockSpec((1, 1), lambda i: (0, 0)),
            pl.BlockSpec((1, 1), lambda i: (0, 0)),
            pl.BlockSpec((1, 1), lambda i: (0, 0)),
        ],
        out_specs=pl.BlockSpec((1, 1), lambda i: (0, 0)),
        out_shape=jax.ShapeDtypeStruct((1, 1), jnp.float32),
    )(coords, mat, W1c, w1m, b1e.reshape(1, 128), W2e.reshape(1, 128),
      b2e.reshape(1, 1), ge.reshape(1, 1), be.reshape(1, 1))
    return ie


def kernel(x, node_mass, element_to_nodes, element_materials,
           W1n, b1n, W2n, b2n, gn, bn, W1e, b1e, W2e, b2e, ge, be):
    n = x.shape[0]
    e = element_to_nodes.shape[0]
    feat = x[:, :, -1]

    latent, ke = _node_stage(feat, node_mass, W1n, b1n, W2n, b2n, gn, bn,
                             bn_rows=5000)

    nc, ns = _sc_geometry()
    nw = nc * ns
    ce = 32
    quantum = nw * ce * 4
    e_pad = ((e + quantum - 1) // quantum) * quantum
    idx_flat = jnp.pad(element_to_nodes.reshape(-1), (0, 4 * (e_pad - e)))
    coords = _gather_sum(latent, idx_flat, e_pad, ce)

    mat_pad = jnp.pad(element_materials, ((0, e_pad - e), (0, 0)))
    ie = _elem_stage(coords, mat_pad, W1e, b1e, W2e, b2e, ge, be,
                     be_rows=2048)
    return (ke[0, 0], ie[0, 0])


# speedup vs baseline: 1.0801x; 1.0801x over previous
import functools

import jax
import jax.numpy as jnp
from jax import lax
from jax.experimental import pallas as pl
from jax.experimental.pallas import tpu as pltpu
from jax.experimental.pallas import tpu_sc as plsc


def _node_kernel(feat_ref, mass_ref, W1n_ref, b1n_ref, W2n_ref, b2n_ref,
                 gn_ref, bn_ref, latent_ref, ke_ref):
    feat = feat_ref[...]
    pos = feat[:, 0:3]
    v = feat[:, 3:6]
    mass = mass_ref[...]
    W1n = W1n_ref[...]
    h1 = jnp.maximum(pos[:, 0:1] * W1n[0:1, :] + pos[:, 1:2] * W1n[1:2, :]
                     + pos[:, 2:3] * W1n[2:3, :] + b1n_ref[...], 0.0)
    h = jnp.dot(h1, W2n_ref[...], preferred_element_type=jnp.float32) + b2n_ref[...]
    mu = jnp.mean(h, axis=-1, keepdims=True)
    var = jnp.mean((h - mu) ** 2, axis=-1, keepdims=True)
    latent_ref[...] = (h - mu) / jnp.sqrt(var + 1e-5) * gn_ref[...] + bn_ref[...]
    ke_part = 0.5 * jnp.sum(mass * jnp.sum(v * v, axis=-1, keepdims=True), keepdims=True)

    @pl.when(pl.program_id(0) == 0)
    def _():
        ke_ref[...] = jnp.zeros_like(ke_ref)

    ke_ref[...] += ke_part


def _node_stage(feat, mass, W1n, b1n, W2n, b2n, gn, bn, bn_rows):
    n = feat.shape[0]
    grid = (n // bn_rows,)
    latent, ke = pl.pallas_call(
        _node_kernel,
        grid=grid,
        in_specs=[
            pl.BlockSpec((bn_rows, 6), lambda i: (i, 0)),
            pl.BlockSpec((bn_rows, 1), lambda i: (i, 0)),
            pl.BlockSpec((3, 128), lambda i: (0, 0)),
            pl.BlockSpec((1, 128), lambda i: (0, 0)),
            pl.BlockSpec((128, 128), lambda i: (0, 0)),
            pl.BlockSpec((1, 128), lambda i: (0, 0)),
            pl.BlockSpec((1, 128), lambda i: (0, 0)),
            pl.BlockSpec((1, 128), lambda i: (0, 0)),
        ],
        out_specs=[
            pl.BlockSpec((bn_rows, 128), lambda i: (i, 0)),
            pl.BlockSpec((1, 1), lambda i: (0, 0)),
        ],
        out_shape=[
            jax.ShapeDtypeStruct((n, 128), jnp.float32),
            jax.ShapeDtypeStruct((1, 1), jnp.float32),
        ],
    )(feat, mass, W1n, b1n.reshape(1, 128), W2n, b2n.reshape(1, 128),
      gn.reshape(1, 128), bn.reshape(1, 128))
    return latent, ke


def _sc_geometry():
    try:
        info = plsc.get_sparse_core_info()
        return info.num_cores, info.num_subcores
    except Exception:
        return 2, 16


def _gather_sum(latent, idx_flat, e_pad, ce):
    nc, ns = _sc_geometry()
    nw = nc * ns
    e_per_w = e_pad // nw
    n_chunks = e_per_w // ce
    mesh = plsc.VectorSubcoreMesh(core_axis_name="c", subcore_axis_name="s",
                                  num_cores=nc, num_subcores=ns)

    nbuf = 4
    @functools.partial(
        pl.kernel, mesh=mesh,
        out_type=jax.ShapeDtypeStruct((e_pad, 128), jnp.float32),
        scratch_types=[
            pltpu.VMEM((n_chunks * 4 * ce,), jnp.int32),
            [pltpu.VMEM((4 * ce, 128), jnp.float32)] * nbuf,
            [pltpu.VMEM((ce, 128), jnp.float32)] * nbuf,
            [pltpu.SemaphoreType.DMA] * nbuf,
            [pltpu.SemaphoreType.DMA] * nbuf,
        ],
    )
    def k(table_hbm, idx_hbm, out_hbm, idx_all, rows, outs, gsem, ssem):
        wid = lax.axis_index("s") * nc + lax.axis_index("c")
        cb = wid * n_chunks
        pltpu.sync_copy(idx_hbm.at[pl.ds(cb * 4 * ce, n_chunks * 4 * ce)],
                        idx_all)
        for j in range(nbuf - 1):
            pltpu.async_copy(
                table_hbm.at[idx_all.at[pl.ds(j * 4 * ce, 4 * ce)]],
                rows[j], gsem[j])

        def group_body(g, _):
            for j in range(nbuf):
                ci = nbuf * g + j
                r_cur, gs_cur = rows[j], gsem[j]
                jp = (j + nbuf - 1) % nbuf
                r_pre, gs_pre = rows[jp], gsem[jp]
                o_cur, ss_cur = outs[j], ssem[j]

                @pl.when(ci + nbuf - 1 < n_chunks)
                def _():
                    pltpu.async_copy(
                        table_hbm.at[idx_all.at[pl.ds((ci + nbuf - 1) * 4 * ce,
                                                      4 * ce)]],
                        r_pre, gs_pre)

                pltpu.make_async_copy(
                    table_hbm.at[idx_all.at[pl.ds(ci * 4 * ce, 4 * ce)]],
                    r_cur, gs_cur).wait()

                @pl.when(ci >= nbuf)
                def _():
                    pltpu.make_async_copy(o_cur, out_hbm.at[pl.ds(0, ce)],
                                          ss_cur).wait()

                @plsc.parallel_loop(0, ce, unroll=8)
                def _(ei):
                    b = 4 * ei
                    for c in range(8):
                        col = pl.ds(c * 16, 16)
                        o_cur[ei, col] = (r_cur[b, col] + r_cur[b + 1, col]
                                          + r_cur[b + 2, col]
                                          + r_cur[b + 3, col])

                pltpu.async_copy(o_cur, out_hbm.at[pl.ds((cb + ci) * ce, ce)],
                                 ss_cur)
            return 0

        lax.fori_loop(0, n_chunks // nbuf, group_body, 0)
        for j in range(nbuf):
            pltpu.make_async_copy(outs[j], out_hbm.at[pl.ds(0, ce)],
                                  ssem[j]).wait()

    return k(latent, idx_flat)


def _elem_kernel(coords_ref, mat_ref, W1c_ref, w1m_ref, b1e_ref, W2e_ref,
                 b2e_ref, ge_ref, be_ref, ie_ref):
    coords = coords_ref[...]
    mat = mat_ref[...]
    h1 = jnp.maximum(jnp.dot(coords, W1c_ref[...], preferred_element_type=jnp.float32)
                     + mat * w1m_ref[...] + b1e_ref[...], 0.0)
    he = jnp.dot(h1, W2e_ref[...], preferred_element_type=jnp.float32) + b2e_ref[...]
    mu = jnp.mean(he, axis=-1, keepdims=True)
    var = jnp.mean((he - mu) ** 2, axis=-1, keepdims=True)
    pe = (he - mu) / jnp.sqrt(var + 1e-5) * ge_ref[...] + be_ref[...]
    part = jnp.sum(pe, keepdims=True)

    @pl.when(pl.program_id(0) == 0)
    def _():
        ie_ref[...] = jnp.zeros_like(ie_ref)

    ie_ref[...] += part


def _elem_stage(coords, mat, W1e, b1e, W2e, b2e, ge, be, be_rows):
    e = coords.shape[0]
    grid = (e // be_rows,)
    W1c = W1e[:128, :]
    w1m = W1e[128:129, :]
    ie = pl.pallas_call(
        _elem_kernel,
        grid=grid,
        in_specs=[
            pl.BlockSpec((be_rows, 128), lambda i: (i, 0)),
            pl.BlockSpec((be_rows, 1), lambda i: (i, 0)),
            pl.BlockSpec((128, 128), lambda i: (0, 0)),
            pl.BlockSpec((1, 128), lambda i: (0, 0)),
            pl.BlockSpec((1, 128), lambda i: (0, 0)),
            pl.BlockSpec((128, 1), lambda i: (0, 0)),
            pl.BlockSpec((1, 1), lambda i: (0, 0)),
            pl.BlockSpec((1, 1), lambda i: (0, 0)),
            pl.BlockSpec((1, 1), lambda i: (0, 0)),
        ],
        out_specs=pl.BlockSpec((1, 1), lambda i: (0, 0)),
        out_shape=jax.ShapeDtypeStruct((1, 1), jnp.float32),
    )(coords, mat, W1c, w1m, b1e.reshape(1, 128), W2e,
      b2e.reshape(1, 1), ge.reshape(1, 1), be.reshape(1, 1))
    return ie


def kernel(x, node_mass, element_to_nodes, element_materials,
           W1n, b1n, W2n, b2n, gn, bn, W1e, b1e, W2e, b2e, ge, be):
    n = x.shape[0]
    e = element_to_nodes.shape[0]
    feat = x[:, :, -1]

    latent, ke = _node_stage(feat, node_mass, W1n, b1n, W2n, b2n, gn, bn,
                             bn_rows=5000)

    nc, ns = _sc_geometry()
    nw = nc * ns
    ce = 32
    quantum = nw * ce * 4
    e_pad = ((e + quantum - 1) // quantum) * quantum
    idx_flat = jnp.pad(element_to_nodes.reshape(-1), (0, 4 * (e_pad - e)))
    coords = _gather_sum(latent, idx_flat, e_pad, ce)

    mat_pad = jnp.pad(element_materials, ((0, e_pad - e), (0, 0)))
    ie = _elem_stage(coords, mat_pad, W1e, b1e, W2e, b2e, ge, be,
                     be_rows=2048)
    return (ke[0, 0], ie[0, 0])


# speedup vs baseline: 96.3406x; 89.1953x over previous
import jax
import jax.numpy as jnp
from jax.experimental import pallas as pl


def _energy_kernel(ft_ref, mass_ref, e_be_ref, ke_ref, ie_ref):
    vx = ft_ref[3:4, :]
    vy = ft_ref[4:5, :]
    vz = ft_ref[5:6, :]
    mass = mass_ref[...]
    sq = vx * vx + vy * vy + vz * vz
    ke_ref[...] = 0.5 * jnp.sum(mass * sq, keepdims=True)
    ie_ref[...] = e_be_ref[...]


def kernel(x, node_mass, element_to_nodes, element_materials,
           W1n, b1n, W2n, b2n, gn, bn, W1e, b1e, W2e, b2e, ge, be):
    e = element_to_nodes.shape[0]
    feat_t = x[:, :, -1].T
    mass_t = node_mass.T
    e_be = (jnp.float32(e) * be).reshape(1, 1)
    ke, ie = pl.pallas_call(
        _energy_kernel,
        out_shape=(
            jax.ShapeDtypeStruct((1, 1), jnp.float32),
            jax.ShapeDtypeStruct((1, 1), jnp.float32),
        ),
    )(feat_t, mass_t, e_be)
    return (ke[0, 0], ie[0, 0])
